# baseline (speedup 1.0000x reference)
import dataclasses
import functools

import jax
import jax.numpy as jnp
from jax import lax
from jax.experimental import pallas as pl
from jax.experimental.pallas import tpu as pltpu
from jax.experimental.pallas import tpu_sc as plsc

_N = 4096
_F = 256
_C = 40
_NB1 = 25
_NB2 = 10
_BMF = 1024
_BMS = 256
_CK = 256
_WFAST = 256


_SCWORKERS = 32
_SCROWS = _N // _SCWORKERS
_SCB = 8
_SCNB = _SCROWS // _SCB
_SCL = 16


def _sc_scan_body(a_hbm, rank_hbm, abuf, rbuf, sa0, sa1, sr0, sr1):
    cix = lax.axis_index("c")
    six = lax.axis_index("s")
    wid = six * 2 + cix
    base = wid * _SCROWS

    def in_copy(batch, buf, sem):
        row0 = base + batch * _SCB
        return pltpu.make_async_copy(
            a_hbm.at[pl.ds(row0, _SCB), pl.ds(0, _WFAST)], abuf.at[buf], sem)

    def out_copy(batch, buf, sem):
        row0 = base + batch * _SCB
        return pltpu.make_async_copy(
            rbuf.at[buf], rank_hbm.at[pl.ds(row0, _SCB), pl.ds(0, _WFAST)],
            sem)

    in_copy(0, 0, sa0).start()
    in_copy(1, 1, sa1).start()

    @pl.loop(0, _SCNB, step=2)
    def _batches(bi):
        for b in range(2):
            cur = bi + b
            sa = sa0 if b == 0 else sa1
            sr = sr0 if b == 0 else sr1
            in_copy(cur, b, sa).wait()

            @pl.when(cur >= 2)
            def _():
                out_copy(cur - 2, b, sr).wait()

            for r in range(_SCB):
                found = jnp.int32(0)
                for j in range(_WFAST // _SCL):
                    v = abuf[b, r, pl.ds(j * _SCL, _SCL)]
                    m = v != 0.0
                    mi = jnp.where(m, 1, 0)
                    cs = plsc.cumsum(mi)
                    rk = cs + found
                    rkf = jnp.where(m & (rk <= _NB1), rk, 0)
                    rbuf[b, r, pl.ds(j * _SCL, _SCL)] = rkf.astype(jnp.float32)
                    found = found + jnp.sum(mi)

            out_copy(cur, b, sr).start()

            @pl.when(cur + 2 < _SCNB)
            def _():
                in_copy(cur + 2, b, sa).start()

    out_copy(_SCNB - 2, 0, sr0).wait()
    out_copy(_SCNB - 1, 1, sr1).wait()


def _sc_compiler_params():
    cp = pltpu.CompilerParams()
    if "needs_layout_passes" in pltpu.CompilerParams.__dataclass_fields__:
        cp = dataclasses.replace(cp, needs_layout_passes=False)
    return cp


def _sc_scan(A):
    fn = pl.kernel(
        _sc_scan_body,
        out_type=jax.ShapeDtypeStruct((_N, _WFAST), jnp.float32),
        mesh=plsc.VectorSubcoreMesh(core_axis_name="c", subcore_axis_name="s"),
        compiler_params=_sc_compiler_params(),
        scratch_types=[
            pltpu.VMEM((2, _SCB, _WFAST), jnp.float32),
            pltpu.VMEM((2, _SCB, _WFAST), jnp.float32),
            pltpu.SemaphoreType.DMA,
            pltpu.SemaphoreType.DMA,
            pltpu.SemaphoreType.DMA,
            pltpu.SemaphoreType.DMA,
        ],
    )
    return fn(A)


def _lrelu(x):
    return jnp.where(x >= 0, x, 0.01 * x)


def _dot(a, b):
    return jax.lax.dot_general(a, b, (((1,), (0,)), ((), ())),
                               preferred_element_type=jnp.float32)


def _split(x):
    hi = x.astype(jnp.bfloat16)
    lo = (x - hi.astype(jnp.float32)).astype(jnp.bfloat16)
    return hi, lo


def _dot3(x, w):
    xhi, xlo = _split(x)
    whi, wlo = _split(w)
    return _dot(xhi, whi) + (_dot(xhi, wlo) + _dot(xlo, whi))


def _sel_agg(a_ref, xa_ref, bm, width, nb, sel2_ref):
    r = jax.lax.broadcasted_iota(jnp.int32, (_CK, _CK), 0)
    c = jax.lax.broadcasted_iota(jnp.int32, (_CK, _CK), 1)
    tri = (r <= c).astype(jnp.bfloat16)
    carry = jnp.zeros((bm, 1), jnp.float32)
    sel_chunks = []
    for ci in range(width // _CK):
        a_c = a_ref[:, ci * _CK:(ci + 1) * _CK].astype(jnp.bfloat16)
        csum = (_dot(a_c, tri) + carry).astype(jnp.bfloat16)
        sel1 = jnp.where(csum <= jnp.bfloat16(nb), a_c, jnp.bfloat16(0))
        sel2 = jnp.where(csum <= jnp.bfloat16(_NB2), a_c, jnp.bfloat16(0))
        sel2_ref[:, ci * _CK:(ci + 1) * _CK] = sel2.astype(jnp.int8)
        sel_chunks.append(sel1)
        carry = carry + jnp.sum(a_c, axis=1, keepdims=True).astype(jnp.float32)
    sel = sel_chunks[0] if len(sel_chunks) == 1 else jnp.concatenate(
        sel_chunks, axis=1)
    xa = xa_ref[...]
    xhi, xlo = _split(xa)
    acc = _dot(sel, xhi) + _dot(sel, xlo)
    return acc, carry


def _mk_layer1_body(bm, width, dyn_cnt):
    def body(a_ref, xa_ref, xb_ref, wnT_ref, bn_ref, wT_ref, b_ref,
             h_ref, sel2_ref, cnt_ref):
        acc, carry = _sel_agg(a_ref, xa_ref, bm, width, _NB1, sel2_ref)
        if dyn_cnt:
            cnt = jnp.minimum(carry, float(_NB1))
            mean = acc / jnp.maximum(cnt, 1.0)
        else:
            mean = acc * (1.0 / _NB1)
        xj = _lrelu(_dot3(mean, wnT_ref[...]) + bn_ref[...])
        xi = _lrelu(_dot3(xb_ref[...], wT_ref[...]) + b_ref[...])
        if dyn_cnt:
            h_ref[...] = xi + jnp.where(carry > 0, xj, 0.0)
        else:
            h_ref[...] = xi + xj
        cnt_ref[...] = carry
    return body


def _mk_layer1_rank_body(bm):
    def body(rk_ref, xa_ref, xb_ref, wnT_ref, bn_ref, wT_ref, b_ref,
             h_ref, sel2_ref, cnt_ref):
        rk = rk_ref[...]
        sel1f = jnp.where((rk >= 0.5) & (rk <= _NB1 + 0.5), 1.0, 0.0)
        sel2f = jnp.where((rk >= 0.5) & (rk <= _NB2 + 0.5), 1.0, 0.0)
        sel2_ref[...] = sel2f.astype(jnp.int8)
        cnt_ref[...] = jnp.sum(sel1f, axis=1, keepdims=True)
        sel1 = sel1f.astype(jnp.bfloat16)
        xhi, xlo = _split(xa_ref[...])
        acc = _dot(sel1, xhi) + _dot(sel1, xlo)
        mean = acc * (1.0 / _NB1)
        xj = _lrelu(_dot3(mean, wnT_ref[...]) + bn_ref[...])
        xi = _lrelu(_dot3(xb_ref[...], wT_ref[...]) + b_ref[...])
        h_ref[...] = xi + xj
    return body


def _run_layer1_rank(bm, rank, X, Wn1T, bn1, W1T, b1):
    grid = (_N // bm,)
    return pl.pallas_call(
        _mk_layer1_rank_body(bm),
        grid=grid,
        in_specs=[
            pl.BlockSpec((bm, _WFAST), _ROW),
            _full((_WFAST, _F)),
            pl.BlockSpec((bm, _F), _ROW),
            _full((_F, _F)),
            _full((1, _F)),
            _full((_F, _F)),
            _full((1, _F)),
        ],
        out_specs=[
            pl.BlockSpec((bm, _F), _ROW),
            pl.BlockSpec((bm, _WFAST), _ROW),
            pl.BlockSpec((bm, 1), _ROW),
        ],
        out_shape=[
            jax.ShapeDtypeStruct((_N, _F), jnp.float32),
            jax.ShapeDtypeStruct((_N, _WFAST), jnp.int8),
            jax.ShapeDtypeStruct((_N, 1), jnp.float32),
        ],
    )(rank, X, X, Wn1T, bn1, W1T, b1)


def _mk_layer2_body(bm, width, dyn_cnt):
    def body(sel2_ref, ha_ref, hb_ref, cnt_ref, wnT_ref, bn_ref, wT_ref,
             b_ref, w3T_ref, b3_ref, o_ref):
        total = cnt_ref[...]
        sel = sel2_ref[...].astype(jnp.bfloat16)
        ha = ha_ref[...]
        hhi, hlo = _split(ha)
        acc = _dot(sel, hhi) + _dot(sel, hlo)
        if dyn_cnt:
            cnt = jnp.minimum(total, float(_NB2))
            mean = acc / jnp.maximum(cnt, 1.0)
        else:
            mean = acc * (1.0 / _NB2)
        xj = _lrelu(_dot3(mean, wnT_ref[...]) + bn_ref[...])
        xi = _lrelu(_dot3(hb_ref[...], wT_ref[...]) + b_ref[...])
        if dyn_cnt:
            h2 = xi + jnp.where(total > 0, xj, 0.0)
        else:
            h2 = xi + xj
        logits = _dot3(h2, w3T_ref[...]) + b3_ref[...]
        m = jnp.max(logits, axis=1, keepdims=True)
        shifted = logits - m
        lse = jnp.log(jnp.sum(jnp.exp(shifted), axis=1, keepdims=True))
        o_ref[...] = shifted - lse
    return body


def _full(shape):
    return pl.BlockSpec(shape, lambda i: (0, 0))


_ROW = lambda i: (i, 0)


def _run_layer1(bm, width, dyn_cnt, A, X, Wn1T, bn1, W1T, b1):
    grid = (_N // bm,)
    return pl.pallas_call(
        _mk_layer1_body(bm, width, dyn_cnt),
        grid=grid,
        in_specs=[
            pl.BlockSpec((bm, width), _ROW),
            _full((width, _F)),
            pl.BlockSpec((bm, _F), _ROW),
            _full((_F, _F)),
            _full((1, _F)),
            _full((_F, _F)),
            _full((1, _F)),
        ],
        out_specs=[
            pl.BlockSpec((bm, _F), _ROW),
            pl.BlockSpec((bm, width), _ROW),
            pl.BlockSpec((bm, 1), _ROW),
        ],
        out_shape=[
            jax.ShapeDtypeStruct((_N, _F), jnp.float32),
            jax.ShapeDtypeStruct((_N, width), jnp.int8),
            jax.ShapeDtypeStruct((_N, 1), jnp.float32),
        ],
    )(A, X, X, Wn1T, bn1, W1T, b1)


def _run_layer2(bm, width, dyn_cnt, sel2, h, cnt, Wn2T, bn2, W2T, b2, W3T, b3):
    grid = (_N // bm,)
    return pl.pallas_call(
        _mk_layer2_body(bm, width, dyn_cnt),
        grid=grid,
        in_specs=[
            pl.BlockSpec((bm, width), _ROW),
            _full((width, _F)),
            pl.BlockSpec((bm, _F), _ROW),
            pl.BlockSpec((bm, 1), _ROW),
            _full((_F, _F)),
            _full((1, _F)),
            _full((_F, _F)),
            _full((1, _F)),
            _full((_F, _C)),
            _full((1, _C)),
        ],
        out_specs=pl.BlockSpec((bm, _C), _ROW),
        out_shape=jax.ShapeDtypeStruct((_N, _C), jnp.float32),
    )(sel2, h, h, cnt, Wn2T, bn2, W2T, b2, W3T, b3)


def kernel(X, A, Wn1, bn1, W1, b1, Wn2, bn2, W2, b2, W3, b3):
    Wn1T, W1T = Wn1.T, W1.T
    Wn2T, W2T = Wn2.T, W2.T
    W3T = W3.T
    bn1_, b1_ = bn1[None, :], b1[None, :]
    bn2_, b2_ = bn2[None, :], b2[None, :]
    b3_ = b3[None, :]

    rank = _sc_scan(A)
    h, sel2, cnt = _run_layer1_rank(_BMF, rank, X, Wn1T, bn1_, W1T, b1_)
    ok = jnp.all(cnt >= float(_NB1))

    def fast_path(_):
        return _run_layer2(_BMF, _WFAST, False, sel2, h, cnt, Wn2T, bn2_, W2T,
                           b2_, W3T, b3_)

    def slow_path(_):
        hs, sel2s, cnts = _run_layer1(_BMS, _N, True, A, X, Wn1T, bn1_, W1T, b1_)
        return _run_layer2(_BMS, _N, True, sel2s, hs, cnts, Wn2T, bn2_, W2T,
                           b2_, W3T, b3_)

    return jax.lax.cond(ok, fast_path, slow_path, None)


# speedup vs baseline: 1.0512x; 1.0512x over previous
import dataclasses
import functools

import jax
import jax.numpy as jnp
from jax import lax
from jax.experimental import pallas as pl
from jax.experimental.pallas import tpu as pltpu
from jax.experimental.pallas import tpu_sc as plsc

_N = 4096
_F = 256
_C = 40
_NB1 = 25
_NB2 = 10
_BMF = 1024
_BMS = 256
_CK = 256
_WFAST = 128


_SCWORKERS = 32
_SCROWS = _N // _SCWORKERS
_SCB = 8
_SCNB = _SCROWS // _SCB
_SCL = 16


def _sc_scan_body(a_hbm, rank_hbm, abuf, rbuf, sa0, sa1, sr0, sr1):
    cix = lax.axis_index("c")
    six = lax.axis_index("s")
    wid = six * 2 + cix
    base = wid * _SCROWS

    def in_copy(batch, buf, sem):
        row0 = base + batch * _SCB
        return pltpu.make_async_copy(
            a_hbm.at[pl.ds(row0, _SCB), pl.ds(0, _WFAST)], abuf.at[buf], sem)

    def out_copy(batch, buf, sem):
        row0 = base + batch * _SCB
        return pltpu.make_async_copy(
            rbuf.at[buf], rank_hbm.at[pl.ds(row0, _SCB), pl.ds(0, _WFAST)],
            sem)

    in_copy(0, 0, sa0).start()
    in_copy(1, 1, sa1).start()

    @pl.loop(0, _SCNB, step=2)
    def _batches(bi):
        for b in range(2):
            cur = bi + b
            sa = sa0 if b == 0 else sa1
            sr = sr0 if b == 0 else sr1
            in_copy(cur, b, sa).wait()

            @pl.when(cur >= 2)
            def _():
                out_copy(cur - 2, b, sr).wait()

            for r in range(_SCB):
                found = jnp.int32(0)
                for j in range(_WFAST // _SCL):
                    v = abuf[b, r, pl.ds(j * _SCL, _SCL)]
                    m = v != 0.0
                    mi = jnp.where(m, 1, 0)
                    cs = plsc.cumsum(mi)
                    rk = cs + found
                    rkf = jnp.where(m & (rk <= _NB1), rk, 0)
                    rbuf[b, r, pl.ds(j * _SCL, _SCL)] = rkf.astype(jnp.float32)
                    found = found + jnp.sum(mi)

            out_copy(cur, b, sr).start()

            @pl.when(cur + 2 < _SCNB)
            def _():
                in_copy(cur + 2, b, sa).start()

    out_copy(_SCNB - 2, 0, sr0).wait()
    out_copy(_SCNB - 1, 1, sr1).wait()


def _sc_compiler_params():
    cp = pltpu.CompilerParams()
    if "needs_layout_passes" in pltpu.CompilerParams.__dataclass_fields__:
        cp = dataclasses.replace(cp, needs_layout_passes=False)
    return cp


def _sc_scan(A):
    fn = pl.kernel(
        _sc_scan_body,
        out_type=jax.ShapeDtypeStruct((_N, _WFAST), jnp.float32),
        mesh=plsc.VectorSubcoreMesh(core_axis_name="c", subcore_axis_name="s"),
        compiler_params=_sc_compiler_params(),
        scratch_types=[
            pltpu.VMEM((2, _SCB, _WFAST), jnp.float32),
            pltpu.VMEM((2, _SCB, _WFAST), jnp.float32),
            pltpu.SemaphoreType.DMA,
            pltpu.SemaphoreType.DMA,
            pltpu.SemaphoreType.DMA,
            pltpu.SemaphoreType.DMA,
        ],
    )
    return fn(A)


def _lrelu(x):
    return jnp.where(x >= 0, x, 0.01 * x)


def _dot(a, b):
    return jax.lax.dot_general(a, b, (((1,), (0,)), ((), ())),
                               preferred_element_type=jnp.float32)


def _split(x):
    hi = x.astype(jnp.bfloat16)
    lo = (x - hi.astype(jnp.float32)).astype(jnp.bfloat16)
    return hi, lo


def _dot3(x, w):
    xhi, xlo = _split(x)
    whi, wlo = _split(w)
    return _dot(xhi, whi) + (_dot(xhi, wlo) + _dot(xlo, whi))


def _sel_agg(a_ref, xa_ref, bm, width, nb, sel2_ref):
    r = jax.lax.broadcasted_iota(jnp.int32, (_CK, _CK), 0)
    c = jax.lax.broadcasted_iota(jnp.int32, (_CK, _CK), 1)
    tri = (r <= c).astype(jnp.bfloat16)
    carry = jnp.zeros((bm, 1), jnp.float32)
    sel_chunks = []
    for ci in range(width // _CK):
        a_c = a_ref[:, ci * _CK:(ci + 1) * _CK].astype(jnp.bfloat16)
        csum = (_dot(a_c, tri) + carry).astype(jnp.bfloat16)
        sel1 = jnp.where(csum <= jnp.bfloat16(nb), a_c, jnp.bfloat16(0))
        sel2 = jnp.where(csum <= jnp.bfloat16(_NB2), a_c, jnp.bfloat16(0))
        sel2_ref[:, ci * _CK:(ci + 1) * _CK] = sel2.astype(jnp.int8)
        sel_chunks.append(sel1)
        carry = carry + jnp.sum(a_c, axis=1, keepdims=True).astype(jnp.float32)
    sel = sel_chunks[0] if len(sel_chunks) == 1 else jnp.concatenate(
        sel_chunks, axis=1)
    xa = xa_ref[...]
    xhi, xlo = _split(xa)
    acc = _dot(sel, xhi) + _dot(sel, xlo)
    return acc, carry


def _mk_layer1_body(bm, width, dyn_cnt):
    def body(a_ref, xa_ref, xb_ref, wnT_ref, bn_ref, wT_ref, b_ref,
             h_ref, sel2_ref, cnt_ref):
        acc, carry = _sel_agg(a_ref, xa_ref, bm, width, _NB1, sel2_ref)
        if dyn_cnt:
            cnt = jnp.minimum(carry, float(_NB1))
            mean = acc / jnp.maximum(cnt, 1.0)
        else:
            mean = acc * (1.0 / _NB1)
        xj = _lrelu(_dot3(mean, wnT_ref[...]) + bn_ref[...])
        xi = _lrelu(_dot3(xb_ref[...], wT_ref[...]) + b_ref[...])
        if dyn_cnt:
            h_ref[...] = xi + jnp.where(carry > 0, xj, 0.0)
        else:
            h_ref[...] = xi + xj
        cnt_ref[...] = carry
    return body


def _mk_layer1_rank_body(bm):
    def body(rk_ref, xa_ref, xi_ref, wnT_ref, bn_ref,
             h_ref, sel2_ref, cnt_ref):
        rk = rk_ref[...]
        sel1f = jnp.where((rk >= 0.5) & (rk <= _NB1 + 0.5), 1.0, 0.0)
        sel2f = jnp.where((rk >= 0.5) & (rk <= _NB2 + 0.5), 1.0, 0.0)
        sel2_ref[...] = sel2f.astype(jnp.int8)
        cnt_ref[...] = jnp.sum(sel1f, axis=1, keepdims=True)
        sel1 = sel1f.astype(jnp.bfloat16)
        xhi, xlo = _split(xa_ref[...])
        acc = _dot(sel1, xhi) + _dot(sel1, xlo)
        mean = acc * (1.0 / _NB1)
        xj = _lrelu(_dot3(mean, wnT_ref[...]) + bn_ref[...])
        h_ref[...] = xi_ref[...] + xj
    return body


def _run_layer1_rank(bm, rank, X, xi, Wn1T, bn1):
    grid = (_N // bm,)
    return pl.pallas_call(
        _mk_layer1_rank_body(bm),
        grid=grid,
        in_specs=[
            pl.BlockSpec((bm, _WFAST), _ROW),
            _full((_WFAST, _F)),
            pl.BlockSpec((bm, _F), _ROW),
            _full((_F, _F)),
            _full((1, _F)),
        ],
        out_specs=[
            pl.BlockSpec((bm, _F), _ROW),
            pl.BlockSpec((bm, _WFAST), _ROW),
            pl.BlockSpec((bm, 1), _ROW),
        ],
        out_shape=[
            jax.ShapeDtypeStruct((_N, _F), jnp.float32),
            jax.ShapeDtypeStruct((_N, _WFAST), jnp.int8),
            jax.ShapeDtypeStruct((_N, 1), jnp.float32),
        ],
    )(rank, X, xi, Wn1T, bn1)


def _xi_body(xb_ref, wT_ref, b_ref, o_ref):
    o_ref[...] = _lrelu(_dot3(xb_ref[...], wT_ref[...]) + b_ref[...])


def _run_xi(bm, X, WT, b):
    grid = (_N // bm,)
    return pl.pallas_call(
        _xi_body,
        grid=grid,
        in_specs=[
            pl.BlockSpec((bm, _F), _ROW),
            _full((_F, _F)),
            _full((1, _F)),
        ],
        out_specs=pl.BlockSpec((bm, _F), _ROW),
        out_shape=jax.ShapeDtypeStruct((_N, _F), jnp.float32),
    )(X, WT, b)


def _mk_layer2_body(bm, width, dyn_cnt):
    def body(sel2_ref, ha_ref, hb_ref, cnt_ref, wnT_ref, bn_ref, wT_ref,
             b_ref, w3T_ref, b3_ref, o_ref):
        total = cnt_ref[...]
        sel = sel2_ref[...].astype(jnp.bfloat16)
        ha = ha_ref[...]
        hhi, hlo = _split(ha)
        acc = _dot(sel, hhi) + _dot(sel, hlo)
        if dyn_cnt:
            cnt = jnp.minimum(total, float(_NB2))
            mean = acc / jnp.maximum(cnt, 1.0)
        else:
            mean = acc * (1.0 / _NB2)
        xj = _lrelu(_dot3(mean, wnT_ref[...]) + bn_ref[...])
        xi = _lrelu(_dot3(hb_ref[...], wT_ref[...]) + b_ref[...])
        if dyn_cnt:
            h2 = xi + jnp.where(total > 0, xj, 0.0)
        else:
            h2 = xi + xj
        logits = _dot3(h2, w3T_ref[...]) + b3_ref[...]
        m = jnp.max(logits, axis=1, keepdims=True)
        shifted = logits - m
        lse = jnp.log(jnp.sum(jnp.exp(shifted), axis=1, keepdims=True))
        o_ref[...] = shifted - lse
    return body


def _full(shape):
    return pl.BlockSpec(shape, lambda i: (0, 0))


_ROW = lambda i: (i, 0)


def _run_layer1(bm, width, dyn_cnt, A, X, Wn1T, bn1, W1T, b1):
    grid = (_N // bm,)
    return pl.pallas_call(
        _mk_layer1_body(bm, width, dyn_cnt),
        grid=grid,
        in_specs=[
            pl.BlockSpec((bm, width), _ROW),
            _full((width, _F)),
            pl.BlockSpec((bm, _F), _ROW),
            _full((_F, _F)),
            _full((1, _F)),
            _full((_F, _F)),
            _full((1, _F)),
        ],
        out_specs=[
            pl.BlockSpec((bm, _F), _ROW),
            pl.BlockSpec((bm, width), _ROW),
            pl.BlockSpec((bm, 1), _ROW),
        ],
        out_shape=[
            jax.ShapeDtypeStruct((_N, _F), jnp.float32),
            jax.ShapeDtypeStruct((_N, width), jnp.int8),
            jax.ShapeDtypeStruct((_N, 1), jnp.float32),
        ],
    )(A, X, X, Wn1T, bn1, W1T, b1)


def _run_layer2(bm, width, dyn_cnt, sel2, h, cnt, Wn2T, bn2, W2T, b2, W3T, b3):
    grid = (_N // bm,)
    return pl.pallas_call(
        _mk_layer2_body(bm, width, dyn_cnt),
        grid=grid,
        in_specs=[
            pl.BlockSpec((bm, width), _ROW),
            _full((width, _F)),
            pl.BlockSpec((bm, _F), _ROW),
            pl.BlockSpec((bm, 1), _ROW),
            _full((_F, _F)),
            _full((1, _F)),
            _full((_F, _F)),
            _full((1, _F)),
            _full((_F, _C)),
            _full((1, _C)),
        ],
        out_specs=pl.BlockSpec((bm, _C), _ROW),
        out_shape=jax.ShapeDtypeStruct((_N, _C), jnp.float32),
    )(sel2, h, h, cnt, Wn2T, bn2, W2T, b2, W3T, b3)


def kernel(X, A, Wn1, bn1, W1, b1, Wn2, bn2, W2, b2, W3, b3):
    Wn1T, W1T = Wn1.T, W1.T
    Wn2T, W2T = Wn2.T, W2.T
    W3T = W3.T
    bn1_, b1_ = bn1[None, :], b1[None, :]
    bn2_, b2_ = bn2[None, :], b2[None, :]
    b3_ = b3[None, :]

    rank = _sc_scan(A)
    xi1 = _run_xi(_BMF, X, W1T, b1_)
    h, sel2, cnt = _run_layer1_rank(_BMF, rank, X, xi1, Wn1T, bn1_)
    ok = jnp.all(cnt >= float(_NB1))

    def fast_path(_):
        return _run_layer2(_BMF, _WFAST, False, sel2, h, cnt, Wn2T, bn2_, W2T,
                           b2_, W3T, b3_)

    def slow_path(_):
        hs, sel2s, cnts = _run_layer1(_BMS, _N, True, A, X, Wn1T, bn1_, W1T, b1_)
        return _run_layer2(_BMS, _N, True, sel2s, hs, cnts, Wn2T, bn2_, W2T,
                           b2_, W3T, b3_)

    return jax.lax.cond(ok, fast_path, slow_path, None)


# speedup vs baseline: 1.0691x; 1.0171x over previous
import dataclasses
import functools

import jax
import jax.numpy as jnp
from jax import lax
from jax.experimental import pallas as pl
from jax.experimental.pallas import tpu as pltpu
from jax.experimental.pallas import tpu_sc as plsc

_N = 4096
_F = 256
_C = 40
_NB1 = 25
_NB2 = 10
_BMF = 1024
_BMS = 256
_CK = 256
_WFAST = 128


_SCWORKERS = 32
_SCROWS = _N // _SCWORKERS
_SCB = 8
_SCNB = _SCROWS // _SCB
_SCL = 16


def _sc_scan_body(a_hbm, rank_hbm, abuf, rbuf, sa0, sa1, sr0, sr1):
    cix = lax.axis_index("c")
    six = lax.axis_index("s")
    wid = six * 2 + cix
    base = wid * _SCROWS

    def in_copy(batch, buf, sem):
        row0 = base + batch * _SCB
        return pltpu.make_async_copy(
            a_hbm.at[pl.ds(row0, _SCB), pl.ds(0, _WFAST)], abuf.at[buf], sem)

    def out_copy(batch, buf, sem):
        row0 = base + batch * _SCB
        return pltpu.make_async_copy(
            rbuf.at[buf], rank_hbm.at[pl.ds(row0, _SCB), pl.ds(0, _WFAST)],
            sem)

    in_copy(0, 0, sa0).start()
    in_copy(1, 1, sa1).start()

    @pl.loop(0, _SCNB, step=2)
    def _batches(bi):
        for b in range(2):
            cur = bi + b
            sa = sa0 if b == 0 else sa1
            sr = sr0 if b == 0 else sr1
            in_copy(cur, b, sa).wait()

            @pl.when(cur >= 2)
            def _():
                out_copy(cur - 2, b, sr).wait()

            for r in range(_SCB):
                found = jnp.float32(0)
                for j in range(_WFAST // _SCL):
                    v = abuf[b, r, pl.ds(j * _SCL, _SCL)]
                    rk = plsc.cumsum(v) + found
                    rkf = jnp.where(rk <= _NB1 + 0.5, rk, 0.0) * v
                    rbuf[b, r, pl.ds(j * _SCL, _SCL)] = rkf
                    found = found + jnp.sum(v)

            out_copy(cur, b, sr).start()

            @pl.when(cur + 2 < _SCNB)
            def _():
                in_copy(cur + 2, b, sa).start()

    out_copy(_SCNB - 2, 0, sr0).wait()
    out_copy(_SCNB - 1, 1, sr1).wait()


def _sc_compiler_params():
    cp = pltpu.CompilerParams()
    if "needs_layout_passes" in pltpu.CompilerParams.__dataclass_fields__:
        cp = dataclasses.replace(cp, needs_layout_passes=False)
    return cp


def _sc_scan(A):
    fn = pl.kernel(
        _sc_scan_body,
        out_type=jax.ShapeDtypeStruct((_N, _WFAST), jnp.float32),
        mesh=plsc.VectorSubcoreMesh(core_axis_name="c", subcore_axis_name="s"),
        compiler_params=_sc_compiler_params(),
        scratch_types=[
            pltpu.VMEM((2, _SCB, _WFAST), jnp.float32),
            pltpu.VMEM((2, _SCB, _WFAST), jnp.float32),
            pltpu.SemaphoreType.DMA,
            pltpu.SemaphoreType.DMA,
            pltpu.SemaphoreType.DMA,
            pltpu.SemaphoreType.DMA,
        ],
    )
    return fn(A)


def _lrelu(x):
    return jnp.where(x >= 0, x, 0.01 * x)


def _dot(a, b):
    return jax.lax.dot_general(a, b, (((1,), (0,)), ((), ())),
                               preferred_element_type=jnp.float32)


def _split(x):
    hi = x.astype(jnp.bfloat16)
    lo = (x - hi.astype(jnp.float32)).astype(jnp.bfloat16)
    return hi, lo


def _dot3(x, w):
    xhi, xlo = _split(x)
    whi, wlo = _split(w)
    return _dot(xhi, whi) + (_dot(xhi, wlo) + _dot(xlo, whi))


def _sel_agg(a_ref, xa_ref, bm, width, nb, sel2_ref):
    r = jax.lax.broadcasted_iota(jnp.int32, (_CK, _CK), 0)
    c = jax.lax.broadcasted_iota(jnp.int32, (_CK, _CK), 1)
    tri = (r <= c).astype(jnp.bfloat16)
    carry = jnp.zeros((bm, 1), jnp.float32)
    sel_chunks = []
    for ci in range(width // _CK):
        a_c = a_ref[:, ci * _CK:(ci + 1) * _CK].astype(jnp.bfloat16)
        csum = (_dot(a_c, tri) + carry).astype(jnp.bfloat16)
        sel1 = jnp.where(csum <= jnp.bfloat16(nb), a_c, jnp.bfloat16(0))
        sel2 = jnp.where(csum <= jnp.bfloat16(_NB2), a_c, jnp.bfloat16(0))
        sel2_ref[:, ci * _CK:(ci + 1) * _CK] = sel2.astype(jnp.int8)
        sel_chunks.append(sel1)
        carry = carry + jnp.sum(a_c, axis=1, keepdims=True).astype(jnp.float32)
    sel = sel_chunks[0] if len(sel_chunks) == 1 else jnp.concatenate(
        sel_chunks, axis=1)
    xa = xa_ref[...]
    xhi, xlo = _split(xa)
    acc = _dot(sel, xhi) + _dot(sel, xlo)
    return acc, carry


def _mk_layer1_body(bm, width, dyn_cnt):
    def body(a_ref, xa_ref, xb_ref, wnT_ref, bn_ref, wT_ref, b_ref,
             h_ref, sel2_ref, cnt_ref):
        acc, carry = _sel_agg(a_ref, xa_ref, bm, width, _NB1, sel2_ref)
        if dyn_cnt:
            cnt = jnp.minimum(carry, float(_NB1))
            mean = acc / jnp.maximum(cnt, 1.0)
        else:
            mean = acc * (1.0 / _NB1)
        xj = _lrelu(_dot3(mean, wnT_ref[...]) + bn_ref[...])
        xi = _lrelu(_dot3(xb_ref[...], wT_ref[...]) + b_ref[...])
        if dyn_cnt:
            h_ref[...] = xi + jnp.where(carry > 0, xj, 0.0)
        else:
            h_ref[...] = xi + xj
        cnt_ref[...] = carry
    return body


def _mk_layer1_rank_body(bm):
    def body(rk_ref, xa_ref, xi_ref, wnT_ref, bn_ref,
             h_ref, sel2_ref, cnt_ref):
        rk = rk_ref[...]
        sel1f = jnp.where((rk >= 0.5) & (rk <= _NB1 + 0.5), 1.0, 0.0)
        sel2f = jnp.where((rk >= 0.5) & (rk <= _NB2 + 0.5), 1.0, 0.0)
        sel2_ref[...] = sel2f.astype(jnp.int8)
        cnt_ref[...] = jnp.sum(sel1f, axis=1, keepdims=True)
        sel1 = sel1f.astype(jnp.bfloat16)
        xhi, xlo = _split(xa_ref[...])
        acc = _dot(sel1, xhi) + _dot(sel1, xlo)
        mean = acc * (1.0 / _NB1)
        xj = _lrelu(_dot3(mean, wnT_ref[...]) + bn_ref[...])
        h_ref[...] = xi_ref[...] + xj
    return body


def _run_layer1_rank(bm, rank, X, xi, Wn1T, bn1):
    grid = (_N // bm,)
    return pl.pallas_call(
        _mk_layer1_rank_body(bm),
        grid=grid,
        in_specs=[
            pl.BlockSpec((bm, _WFAST), _ROW),
            _full((_WFAST, _F)),
            pl.BlockSpec((bm, _F), _ROW),
            _full((_F, _F)),
            _full((1, _F)),
        ],
        out_specs=[
            pl.BlockSpec((bm, _F), _ROW),
            pl.BlockSpec((bm, _WFAST), _ROW),
            pl.BlockSpec((bm, 1), _ROW),
        ],
        out_shape=[
            jax.ShapeDtypeStruct((_N, _F), jnp.float32),
            jax.ShapeDtypeStruct((_N, _WFAST), jnp.int8),
            jax.ShapeDtypeStruct((_N, 1), jnp.float32),
        ],
    )(rank, X, xi, Wn1T, bn1)


def _xi_body(xb_ref, wT_ref, b_ref, o_ref):
    o_ref[...] = _lrelu(_dot3(xb_ref[...], wT_ref[...]) + b_ref[...])


def _run_xi(bm, X, WT, b):
    grid = (_N // bm,)
    return pl.pallas_call(
        _xi_body,
        grid=grid,
        in_specs=[
            pl.BlockSpec((bm, _F), _ROW),
            _full((_F, _F)),
            _full((1, _F)),
        ],
        out_specs=pl.BlockSpec((bm, _F), _ROW),
        out_shape=jax.ShapeDtypeStruct((_N, _F), jnp.float32),
    )(X, WT, b)


def _mk_layer2_body(bm, width, dyn_cnt):
    def body(sel2_ref, ha_ref, hb_ref, cnt_ref, wnT_ref, bn_ref, wT_ref,
             b_ref, w3T_ref, b3_ref, o_ref):
        total = cnt_ref[...]
        sel = sel2_ref[...].astype(jnp.bfloat16)
        ha = ha_ref[...]
        hhi, hlo = _split(ha)
        acc = _dot(sel, hhi) + _dot(sel, hlo)
        if dyn_cnt:
            cnt = jnp.minimum(total, float(_NB2))
            mean = acc / jnp.maximum(cnt, 1.0)
        else:
            mean = acc * (1.0 / _NB2)
        xj = _lrelu(_dot3(mean, wnT_ref[...]) + bn_ref[...])
        xi = _lrelu(_dot3(hb_ref[...], wT_ref[...]) + b_ref[...])
        if dyn_cnt:
            h2 = xi + jnp.where(total > 0, xj, 0.0)
        else:
            h2 = xi + xj
        logits = _dot3(h2, w3T_ref[...]) + b3_ref[...]
        m = jnp.max(logits, axis=1, keepdims=True)
        shifted = logits - m
        lse = jnp.log(jnp.sum(jnp.exp(shifted), axis=1, keepdims=True))
        o_ref[...] = shifted - lse
    return body


def _full(shape):
    return pl.BlockSpec(shape, lambda i: (0, 0))


_ROW = lambda i: (i, 0)


def _run_layer1(bm, width, dyn_cnt, A, X, Wn1T, bn1, W1T, b1):
    grid = (_N // bm,)
    return pl.pallas_call(
        _mk_layer1_body(bm, width, dyn_cnt),
        grid=grid,
        in_specs=[
            pl.BlockSpec((bm, width), _ROW),
            _full((width, _F)),
            pl.BlockSpec((bm, _F), _ROW),
            _full((_F, _F)),
            _full((1, _F)),
            _full((_F, _F)),
            _full((1, _F)),
        ],
        out_specs=[
            pl.BlockSpec((bm, _F), _ROW),
            pl.BlockSpec((bm, width), _ROW),
            pl.BlockSpec((bm, 1), _ROW),
        ],
        out_shape=[
            jax.ShapeDtypeStruct((_N, _F), jnp.float32),
            jax.ShapeDtypeStruct((_N, width), jnp.int8),
            jax.ShapeDtypeStruct((_N, 1), jnp.float32),
        ],
    )(A, X, X, Wn1T, bn1, W1T, b1)


def _run_layer2(bm, width, dyn_cnt, sel2, h, cnt, Wn2T, bn2, W2T, b2, W3T, b3):
    grid = (_N // bm,)
    return pl.pallas_call(
        _mk_layer2_body(bm, width, dyn_cnt),
        grid=grid,
        in_specs=[
            pl.BlockSpec((bm, width), _ROW),
            _full((width, _F)),
            pl.BlockSpec((bm, _F), _ROW),
            pl.BlockSpec((bm, 1), _ROW),
            _full((_F, _F)),
            _full((1, _F)),
            _full((_F, _F)),
            _full((1, _F)),
            _full((_F, _C)),
            _full((1, _C)),
        ],
        out_specs=pl.BlockSpec((bm, _C), _ROW),
        out_shape=jax.ShapeDtypeStruct((_N, _C), jnp.float32),
    )(sel2, h, h, cnt, Wn2T, bn2, W2T, b2, W3T, b3)


def kernel(X, A, Wn1, bn1, W1, b1, Wn2, bn2, W2, b2, W3, b3):
    Wn1T, W1T = Wn1.T, W1.T
    Wn2T, W2T = Wn2.T, W2.T
    W3T = W3.T
    bn1_, b1_ = bn1[None, :], b1[None, :]
    bn2_, b2_ = bn2[None, :], b2[None, :]
    b3_ = b3[None, :]

    rank = _sc_scan(A)
    xi1 = _run_xi(_BMF, X, W1T, b1_)
    h, sel2, cnt = _run_layer1_rank(_BMF, rank, X, xi1, Wn1T, bn1_)
    ok = jnp.all(cnt >= float(_NB1))

    def fast_path(_):
        return _run_layer2(_BMF, _WFAST, False, sel2, h, cnt, Wn2T, bn2_, W2T,
                           b2_, W3T, b3_)

    def slow_path(_):
        hs, sel2s, cnts = _run_layer1(_BMS, _N, True, A, X, Wn1T, bn1_, W1T, b1_)
        return _run_layer2(_BMS, _N, True, sel2s, hs, cnts, Wn2T, bn2_, W2T,
                           b2_, W3T, b3_)

    return jax.lax.cond(ok, fast_path, slow_path, None)
